# baseline (speedup 1.0000x reference)
import functools

import jax
import jax.numpy as jnp
from jax import lax
from jax.experimental import pallas as pl
from jax.experimental.pallas import tpu as pltpu
from jax.experimental.pallas import tpu_sc as plsc

N = 10000
E = 320000
D = 128
KG = 3
PDIM = 4
EPS = 1e-15

NC = 2
NS = 16
NW = NC * NS
LANES = 16
CHUNK = 64
NCHUNKS = E // CHUNK
BASE_CHUNKS = NCHUNKS // NW
EXTRA = NCHUNKS - BASE_CHUNKS * NW
NPAD = 10240
ROWS_SUB = NPAD // NS
ZROWS = 128


def _tc_pre_body(vals_ref, g_ref, root_ref, dense_ref, bias_ref, pseudo_ref,
                 mu0_ref, s0_ref, mu1_ref, s1_ref,
                 xg_ref, r_ref, w0_ref, w1_ref):
    x = vals_ref[...]
    xg_ref[...] = jnp.dot(x, g_ref[...], preferred_element_type=jnp.float32)
    r_ref[...] = (jnp.dot(x, root_ref[...] + dense_ref[...],
                          preferred_element_type=jnp.float32) + bias_ref[...])
    for mu_ref, s_ref, w_ref in ((mu0_ref, s0_ref, w0_ref),
                                 (mu1_ref, s1_ref, w1_ref)):
        mu = mu_ref[...]
        sg = s_ref[...]
        for k in range(KG):
            acc = None
            for dd in range(PDIM):
                pd = pseudo_ref[dd]
                mkd = mu[k:k + 1, dd:dd + 1]
                skd = sg[k:k + 1, dd:dd + 1]
                t = (pd - mkd) ** 2 * (-0.5 / (EPS + skd * skd))
                acc = t if acc is None else acc + t
            w_ref[k] = jnp.exp(acc)


_tc_pre = pl.pallas_call(
    _tc_pre_body,
    out_shape=[
        jax.ShapeDtypeStruct((N, KG * D), jnp.float32),
        jax.ShapeDtypeStruct((N, D), jnp.float32),
        jax.ShapeDtypeStruct((KG, E // 128, 128), jnp.float32),
        jax.ShapeDtypeStruct((KG, E // 128, 128), jnp.float32),
    ],
)


def _combine_bn(agg_ref, cnt_ref, r_ref, gamma_ref, beta_ref):
    agg = agg_ref[0, 0:N] + agg_ref[1, 0:N]
    cnt = cnt_ref[0, 0:N, 0:1] + cnt_ref[1, 0:N, 0:1]
    y = agg / jnp.maximum(cnt, 1.0) + r_ref[...]
    m = jnp.mean(y, axis=0, keepdims=True)
    v = jnp.mean((y - m) ** 2, axis=0, keepdims=True)
    return gamma_ref[...] * ((y - m) / jnp.sqrt(v + 1e-5)) + beta_ref[...]


def _tc_mid_body(agg_ref, cnt_ref, r_ref, gamma_ref, beta_ref,
                 g_ref, root_ref, dense_ref, bias_ref, xg_ref, rout_ref):
    y = _combine_bn(agg_ref, cnt_ref, r_ref, gamma_ref, beta_ref)
    x1 = jnp.maximum(y, 0.0)
    xg_ref[...] = jnp.dot(x1, g_ref[...], preferred_element_type=jnp.float32)
    rout_ref[...] = (jnp.dot(x1, root_ref[...] + dense_ref[...],
                             preferred_element_type=jnp.float32) + bias_ref[...])


_tc_mid = pl.pallas_call(
    _tc_mid_body,
    out_shape=[
        jax.ShapeDtypeStruct((N, KG * D), jnp.float32),
        jax.ShapeDtypeStruct((N, D), jnp.float32),
    ],
)


def _tc_post_body(agg_ref, cnt_ref, r_ref, gamma_ref, beta_ref, out_ref):
    out_ref[...] = _combine_bn(agg_ref, cnt_ref, r_ref, gamma_ref, beta_ref)


_tc_post = pl.pallas_call(
    _tc_post_body,
    out_shape=jax.ShapeDtypeStruct((N, D), jnp.float32),
)


_SC_MESH = plsc.VectorSubcoreMesh(core_axis_name="c", subcore_axis_name="s")


@functools.partial(
    pl.kernel,
    out_type=[
        jax.ShapeDtypeStruct((NC, NPAD, D), jnp.float32),
        jax.ShapeDtypeStruct((NC, NPAD, LANES), jnp.float32),
    ],
    mesh=_SC_MESH,
    compiler_params=pltpu.CompilerParams(use_tc_tiling_on_sc=False,
                                         needs_layout_passes=False),
    scratch_types=[
        pltpu.VMEM((CHUNK,), jnp.int32),
        pltpu.VMEM((CHUNK,), jnp.int32),
        pltpu.VMEM((KG, CHUNK), jnp.float32),
        pltpu.VMEM((CHUNK, KG * D), jnp.float32),
        pltpu.VMEM((CHUNK, D), jnp.float32),
        pltpu.VMEM((CHUNK, LANES), jnp.float32),
        pltpu.VMEM((ZROWS, LANES), jnp.float32),
        pltpu.VMEM_SHARED((NPAD, D), jnp.float32),
        pltpu.VMEM_SHARED((NPAD, LANES), jnp.float32),
        pltpu.SemaphoreType.DMA,
    ],
)
def _sc_conv(xg_hbm, src_hbm, dst_hbm, w_hbm, agg_out, cnt_out,
             src_v, dst_v, w_v, rows_v, msg_v, ones_v, zc_v, agg_sh, cnt_sh,
             sem):
    c = lax.axis_index("c")
    s = lax.axis_index("s")
    wid = s * NC + c

    zf = jnp.zeros((LANES,), jnp.float32)
    of = jnp.ones((LANES,), jnp.float32)

    def zmsg(i, t):
        for j in range(D // LANES):
            msg_v[i, pl.ds(j * LANES, LANES)] = zf
        ones_v[i, pl.ds(0, LANES)] = of
        return t

    lax.fori_loop(0, CHUNK, zmsg, 0)

    def zzc(i, t):
        zc_v[i, pl.ds(0, LANES)] = zf
        return t

    lax.fori_loop(0, ZROWS, zzc, 0)

    row0 = s * ROWS_SUB
    for j in range(ROWS_SUB // CHUNK):
        pltpu.sync_copy(msg_v,
                        agg_sh.at[pl.ds(row0 + j * CHUNK, CHUNK)])
    for j in range(ROWS_SUB // ZROWS):
        pltpu.sync_copy(zc_v, cnt_sh.at[pl.ds(row0 + j * ZROWS, ZROWS)])
    plsc.subcore_barrier()

    eids = lax.broadcasted_iota(jnp.int32, (LANES,), 0)
    nmine = jnp.where(wid < EXTRA, BASE_CHUNKS + 1, BASE_CHUNKS)

    def chunk_body(i, t):
        chunk = wid + i * NW
        base = chunk * CHUNK
        pltpu.sync_copy(src_hbm.at[pl.ds(base, CHUNK)], src_v)
        pltpu.sync_copy(dst_hbm.at[pl.ds(base, CHUNK)], dst_v)
        for k in range(KG):
            pltpu.sync_copy(w_hbm.at[k, pl.ds(chunk, 1), :],
                            w_v.at[pl.ds(k, 1)])
        pltpu.async_copy(xg_hbm.at[src_v], rows_v, sem).wait()

        egs = [eids + g * LANES for g in range(CHUNK // LANES)]
        wks = [[w_v[k, pl.ds(g * LANES, LANES)] for k in range(KG)]
               for g in range(CHUNK // LANES)]

        def fbody(f, t2):
            for g in range(CHUNK // LANES):
                acc = None
                for k in range(KG):
                    col = jnp.full((LANES,), 0, jnp.int32) + (f + k * D)
                    v16 = plsc.load_gather(rows_v, [egs[g], col])
                    term = v16 * wks[g][k]
                    acc = term if acc is None else acc + term
                fcol = jnp.full((LANES,), 0, jnp.int32) + f
                plsc.store_scatter(msg_v, [egs[g], fcol], acc)
            return t2

        lax.fori_loop(0, D, fbody, 0)

        pltpu.sync_copy(msg_v, agg_sh.at[dst_v], add=True)
        pltpu.sync_copy(ones_v, cnt_sh.at[dst_v], add=True)
        return t

    lax.fori_loop(0, nmine, chunk_body, 0)

    plsc.subcore_barrier()
    pltpu.sync_copy(agg_sh.at[pl.ds(row0, ROWS_SUB)],
                    agg_out.at[c, pl.ds(row0, ROWS_SUB)])
    pltpu.sync_copy(cnt_sh.at[pl.ds(row0, ROWS_SUB)],
                    cnt_out.at[c, pl.ds(row0, ROWS_SUB)])


def kernel(vals, edges, pseudo, g0, mu0, sigma0, root0, bias0, dense0,
           gamma0, beta0, g1, mu1, sigma1, root1, bias1, dense1, gamma1,
           beta1):
    src = edges[0]
    dst = edges[1]
    pseudo_t = pseudo.T.reshape(PDIM, E // 128, 128)
    xg0, r0, w0, w1 = _tc_pre(vals, g0, root0, dense0, bias0, pseudo_t,
                              mu0, sigma0, mu1, sigma1)
    agg0, cnt0 = _sc_conv(xg0, src, dst, w0.reshape(KG, NCHUNKS, CHUNK))
    xg1, r1 = _tc_mid(agg0, cnt0, r0, gamma0, beta0,
                      g1, root1, dense1, bias1)
    agg1, cnt1 = _sc_conv(xg1, src, dst, w1.reshape(KG, NCHUNKS, CHUNK))
    return _tc_post(agg1, cnt1, r1, gamma1, beta1)


# speedup vs baseline: 1.4181x; 1.4181x over previous
import functools

import jax
import jax.numpy as jnp
from jax import lax
from jax.experimental import pallas as pl
from jax.experimental.pallas import tpu as pltpu
from jax.experimental.pallas import tpu_sc as plsc

N = 10000
E = 320000
D = 128
KG = 3
PDIM = 4
EPS = 1e-15

NC = 2
NS = 16
LANES = 16
DH = D // NC
GCOLS = KG * DH
MW = DH + LANES
CHUNK = 80
NCHUNKS = E // CHUNK
CPT = NCHUNKS // NS
SUP = 10
NSUP = CPT // SUP
NPAD = 10240
ROWS_SUB = NPAD // NS
EGROUPS = CHUNK // LANES


def _tc_pre_body(vals_ref, gp_ref, root_ref, dense_ref, bias_ref, pseudo_ref,
                 mu0_ref, s0_ref, mu1_ref, s1_ref,
                 xg_ref, r_ref, w0_ref, w1_ref):
    x = vals_ref[...]
    xgfull = jnp.dot(x, gp_ref[...], preferred_element_type=jnp.float32)
    xg_ref[0] = xgfull[:, 0:GCOLS]
    xg_ref[1] = xgfull[:, GCOLS:2 * GCOLS]
    r_ref[...] = (jnp.dot(x, root_ref[...] + dense_ref[...],
                          preferred_element_type=jnp.float32) + bias_ref[...])
    for mu_ref, s_ref, w_ref in ((mu0_ref, s0_ref, w0_ref),
                                 (mu1_ref, s1_ref, w1_ref)):
        mu = mu_ref[...]
        sg = s_ref[...]
        for k in range(KG):
            acc = None
            for dd in range(PDIM):
                pd = pseudo_ref[dd]
                mkd = mu[k:k + 1, dd:dd + 1]
                skd = sg[k:k + 1, dd:dd + 1]
                t = (pd - mkd) ** 2 * (-0.5 / (EPS + skd * skd))
                acc = t if acc is None else acc + t
            w_ref[k] = jnp.exp(acc)


_tc_pre = pl.pallas_call(
    _tc_pre_body,
    out_shape=[
        jax.ShapeDtypeStruct((NC, N, GCOLS), jnp.float32),
        jax.ShapeDtypeStruct((N, D), jnp.float32),
        jax.ShapeDtypeStruct((KG, E // 128, 128), jnp.float32),
        jax.ShapeDtypeStruct((KG, E // 128, 128), jnp.float32),
    ],
)


def _combine_bn(agg_ref, r_ref, gamma_ref, beta_ref):
    feat = jnp.concatenate(
        [agg_ref[0:N, 0:DH], agg_ref[0:N, MW:MW + DH]], axis=1)
    cnt = agg_ref[0:N, DH:DH + 1]
    y = feat / jnp.maximum(cnt, 1.0) + r_ref[...]
    m = jnp.mean(y, axis=0, keepdims=True)
    v = jnp.mean((y - m) ** 2, axis=0, keepdims=True)
    return gamma_ref[...] * ((y - m) / jnp.sqrt(v + 1e-5)) + beta_ref[...]


def _tc_mid_body(agg_ref, r_ref, gamma_ref, beta_ref,
                 gp_ref, root_ref, dense_ref, bias_ref, xg_ref, rout_ref):
    y = _combine_bn(agg_ref, r_ref, gamma_ref, beta_ref)
    x1 = jnp.maximum(y, 0.0)
    xgfull = jnp.dot(x1, gp_ref[...], preferred_element_type=jnp.float32)
    xg_ref[0] = xgfull[:, 0:GCOLS]
    xg_ref[1] = xgfull[:, GCOLS:2 * GCOLS]
    rout_ref[...] = (jnp.dot(x1, root_ref[...] + dense_ref[...],
                             preferred_element_type=jnp.float32) + bias_ref[...])


_tc_mid = pl.pallas_call(
    _tc_mid_body,
    out_shape=[
        jax.ShapeDtypeStruct((NC, N, GCOLS), jnp.float32),
        jax.ShapeDtypeStruct((N, D), jnp.float32),
    ],
)


def _tc_post_body(agg_ref, r_ref, gamma_ref, beta_ref, out_ref):
    out_ref[...] = _combine_bn(agg_ref, r_ref, gamma_ref, beta_ref)


_tc_post = pl.pallas_call(
    _tc_post_body,
    out_shape=jax.ShapeDtypeStruct((N, D), jnp.float32),
)


_SC_MESH = plsc.VectorSubcoreMesh(core_axis_name="c", subcore_axis_name="s")


@functools.partial(
    pl.kernel,
    out_type=jax.ShapeDtypeStruct((NPAD, NC * MW), jnp.float32),
    mesh=_SC_MESH,
    compiler_params=pltpu.CompilerParams(use_tc_tiling_on_sc=False,
                                         needs_layout_passes=False),
    scratch_types=[
        pltpu.VMEM((2, SUP, CHUNK), jnp.int32),
        pltpu.VMEM((2, SUP, CHUNK), jnp.int32),
        pltpu.VMEM((2, KG, SUP, CHUNK), jnp.float32),
        pltpu.VMEM((3, CHUNK, GCOLS), jnp.float32),
        pltpu.VMEM((2, CHUNK, MW), jnp.float32),
        pltpu.VMEM_SHARED((NPAD, MW), jnp.float32),
        pltpu.SemaphoreType.DMA((2,)),
        pltpu.SemaphoreType.DMA((3,)),
        pltpu.SemaphoreType.DMA((2,)),
    ],
)
def _sc_conv(xg_hbm, srcc_hbm, dstc_hbm, w_hbm, agg_out,
             src_v, dst_v, w_v, rows_v, msg_v, agg_sh,
             sem_i, sem_g, sem_s):
    c = lax.axis_index("c")
    s = lax.axis_index("s")

    zf = jnp.zeros((LANES,), jnp.float32)
    of = jnp.ones((LANES,), jnp.float32)

    def zmsg(i, t):
        for j in range(MW // LANES):
            msg_v[0, i, pl.ds(j * LANES, LANES)] = zf
        return t

    lax.fori_loop(0, CHUNK, zmsg, 0)
    row0 = s * ROWS_SUB
    for j in range(ROWS_SUB // CHUNK):
        pltpu.sync_copy(msg_v.at[0],
                        agg_sh.at[pl.ds(row0 + j * CHUNK, CHUNK)])

    def ones_cols(i, t):
        msg_v[0, i, pl.ds(DH, LANES)] = of
        msg_v[1, i, pl.ds(DH, LANES)] = of
        return t

    lax.fori_loop(0, CHUNK, ones_cols, 0)
    plsc.subcore_barrier()

    eids = lax.broadcasted_iota(jnp.int32, (LANES,), 0)
    egs = [eids + g * LANES for g in range(EGROUPS)]
    chunk0 = s * CPT

    def load_super(sup, buf):
        sl = pl.ds(chunk0 + sup * SUP, SUP)
        pltpu.async_copy(srcc_hbm.at[sl, :], src_v.at[buf], sem_i.at[buf])
        pltpu.async_copy(dstc_hbm.at[sl, :], dst_v.at[buf], sem_i.at[buf])
        pltpu.async_copy(w_hbm.at[:, sl, :], w_v.at[buf], sem_i.at[buf])

    def wait_super(buf):
        pltpu.make_async_copy(srcc_hbm.at[pl.ds(0, SUP), :],
                              src_v.at[buf], sem_i.at[buf]).wait()
        pltpu.make_async_copy(dstc_hbm.at[pl.ds(0, SUP), :],
                              dst_v.at[buf], sem_i.at[buf]).wait()
        pltpu.make_async_copy(w_hbm.at[:, pl.ds(0, SUP), :],
                              w_v.at[buf], sem_i.at[buf]).wait()

    def gather(buf, q, slot):
        pltpu.async_copy(xg_hbm.at[src_v.at[buf, q]], rows_v.at[slot],
                         sem_g.at[slot])

    def wait_gather(slot):
        pltpu.make_async_copy(xg_hbm.at[src_v.at[0, 0]], rows_v.at[slot],
                              sem_g.at[slot]).wait()

    load_super(0, 0)

    def super_body(sup, t):
        cur = lax.rem(sup, 2)
        nxt = lax.rem(sup + 1, 2)
        wait_super(cur)

        coff = lax.broadcast(c * N, (LANES,))

        def rebase(i, t2):
            for g in range(EGROUPS):
                sl = pl.ds(g * LANES, LANES)
                src_v[cur, i, sl] = src_v[cur, i, sl] + coff
            return t2

        lax.fori_loop(0, SUP, rebase, 0)

        @pl.when(sup < NSUP - 1)
        def _():
            load_super(sup + 1, nxt)

        base_slot = lax.rem(sup * SUP, 3)
        for q in range(SUP):
            if q < 3:
                gather(cur, q, lax.rem(base_slot + q, 3))

        for q in range(SUP):
            slot = lax.rem(base_slot + q, 3)
            slot_s = lax.broadcast(slot, (LANES,))
            mb = q % 2
            wait_gather(slot)

            wks = [[w_v[cur, k, q, pl.ds(g * LANES, LANES)]
                    for k in range(KG)] for g in range(EGROUPS)]

            if q >= 2:
                pltpu.make_async_copy(
                    msg_v.at[mb], agg_sh.at[dst_v.at[cur, q]],
                    sem_s.at[mb]).wait()
            else:
                @pl.when(sup > 0)
                def _():
                    pltpu.make_async_copy(
                        msg_v.at[mb], agg_sh.at[dst_v.at[cur, q]],
                        sem_s.at[mb]).wait()

            mbs = lax.broadcast(jnp.int32(mb), (LANES,))

            def fbody(f, t2):
                for g in range(EGROUPS):
                    acc = None
                    for k in range(KG):
                        col = lax.broadcast(f + k * DH, (LANES,))
                        v16 = plsc.load_gather(rows_v, [slot_s, egs[g], col])
                        term = v16 * wks[g][k]
                        acc = term if acc is None else acc + term
                    fcol = lax.broadcast(f, (LANES,))
                    plsc.store_scatter(msg_v, [mbs, egs[g], fcol], acc)
                return t2

            lax.fori_loop(0, DH, fbody, 0)

            if q < SUP - 3:
                gather(cur, q + 3, slot)

            pltpu.async_copy(msg_v.at[mb], agg_sh.at[dst_v.at[cur, q]],
                             sem_s.at[mb], add=True)
        return t

    lax.fori_loop(0, NSUP, super_body, 0)

    for mb in range(2):
        pltpu.make_async_copy(msg_v.at[mb], agg_sh.at[dst_v.at[0, 0]],
                              sem_s.at[mb]).wait()
    plsc.subcore_barrier()
    pltpu.sync_copy(agg_sh.at[pl.ds(row0, ROWS_SUB)],
                    agg_out.at[pl.ds(row0, ROWS_SUB), pl.ds(c * MW, MW)])


def kernel(vals, edges, pseudo, g0, mu0, sigma0, root0, bias0, dense0,
           gamma0, beta0, g1, mu1, sigma1, root1, bias1, dense1, gamma1,
           beta1):
    srcc = edges[0].reshape(NCHUNKS, CHUNK)
    dstc = edges[1].reshape(NCHUNKS, CHUNK)
    pseudo_t = pseudo.T.reshape(PDIM, E // 128, 128)

    def permute(g):
        return jnp.concatenate(
            [g[:, k * D + cc * DH:k * D + (cc + 1) * DH]
             for cc in range(NC) for k in range(KG)], axis=1)

    gp0 = permute(g0)
    gp1 = permute(g1)
    xg0, r0, w0, w1 = _tc_pre(vals, gp0, root0, dense0, bias0, pseudo_t,
                              mu0, sigma0, mu1, sigma1)
    agg0 = _sc_conv(xg0.reshape(NC * N, GCOLS), srcc, dstc,
                    w0.reshape(KG, NCHUNKS, CHUNK))
    xg1, r1 = _tc_mid(agg0, r0, gamma0, beta0, gp1, root1, dense1, bias1)
    agg1 = _sc_conv(xg1.reshape(NC * N, GCOLS), srcc, dstc,
                    w1.reshape(KG, NCHUNKS, CHUNK))
    return _tc_post(agg1, r1, gamma1, beta1)


# speedup vs baseline: 5.0850x; 3.5858x over previous
import functools

import jax
import jax.numpy as jnp
from jax import lax
from jax.experimental import pallas as pl
from jax.experimental.pallas import tpu as pltpu
from jax.experimental.pallas import tpu_sc as plsc

N = 10000
E = 320000
D = 128
KG = 3
PDIM = 4
EPS = 1e-15

NC = 2
NS = 16
LANES = 16
DH = D // NC
GCOLS = KG * DH
MW = DH + LANES
CHUNK = 80
NCHUNKS = E // CHUNK
CPT = NCHUNKS // NS
SUP = 10
NSUP = CPT // SUP
NPAD = 10240
ROWS_SUB = NPAD // NS
EGROUPS = CHUNK // LANES


def _tc_pre_body(vals_ref, gp_ref, root_ref, dense_ref, bias_ref, pseudo_ref,
                 mu0_ref, s0_ref, mu1_ref, s1_ref,
                 xg_ref, r_ref, w0_ref, w1_ref):
    x = vals_ref[...]
    xgfull = jnp.dot(x, gp_ref[...], preferred_element_type=jnp.float32)
    xg_ref[0] = xgfull[:, 0:GCOLS]
    xg_ref[1] = xgfull[:, GCOLS:2 * GCOLS]
    r_ref[...] = (jnp.dot(x, root_ref[...] + dense_ref[...],
                          preferred_element_type=jnp.float32) + bias_ref[...])
    for mu_ref, s_ref, w_ref in ((mu0_ref, s0_ref, w0_ref),
                                 (mu1_ref, s1_ref, w1_ref)):
        mu = mu_ref[...]
        sg = s_ref[...]
        for k in range(KG):
            acc = None
            for dd in range(PDIM):
                pd = pseudo_ref[dd]
                mkd = mu[k:k + 1, dd:dd + 1]
                skd = sg[k:k + 1, dd:dd + 1]
                t = (pd - mkd) ** 2 * (-0.5 / (EPS + skd * skd))
                acc = t if acc is None else acc + t
            w_ref[k] = jnp.exp(acc)


_tc_pre = pl.pallas_call(
    _tc_pre_body,
    out_shape=[
        jax.ShapeDtypeStruct((NC, N, GCOLS), jnp.float32),
        jax.ShapeDtypeStruct((N, D), jnp.float32),
        jax.ShapeDtypeStruct((KG, E // 128, 128), jnp.float32),
        jax.ShapeDtypeStruct((KG, E // 128, 128), jnp.float32),
    ],
)


def _combine_bn(agg_ref, r_ref, gamma_ref, beta_ref):
    feat = jnp.concatenate(
        [agg_ref[0:N, 0:DH], agg_ref[0:N, MW:MW + DH]], axis=1)
    cnt = agg_ref[0:N, DH:DH + 1]
    y = feat / jnp.maximum(cnt, 1.0) + r_ref[...]
    m = jnp.mean(y, axis=0, keepdims=True)
    v = jnp.mean((y - m) ** 2, axis=0, keepdims=True)
    return gamma_ref[...] * ((y - m) / jnp.sqrt(v + 1e-5)) + beta_ref[...]


def _tc_mid_body(agg_ref, r_ref, gamma_ref, beta_ref,
                 gp_ref, root_ref, dense_ref, bias_ref, xg_ref, rout_ref):
    y = _combine_bn(agg_ref, r_ref, gamma_ref, beta_ref)
    x1 = jnp.maximum(y, 0.0)
    xgfull = jnp.dot(x1, gp_ref[...], preferred_element_type=jnp.float32)
    xg_ref[0] = xgfull[:, 0:GCOLS]
    xg_ref[1] = xgfull[:, GCOLS:2 * GCOLS]
    rout_ref[...] = (jnp.dot(x1, root_ref[...] + dense_ref[...],
                             preferred_element_type=jnp.float32) + bias_ref[...])


_tc_mid = pl.pallas_call(
    _tc_mid_body,
    out_shape=[
        jax.ShapeDtypeStruct((NC, N, GCOLS), jnp.float32),
        jax.ShapeDtypeStruct((N, D), jnp.float32),
    ],
)


def _tc_post_body(agg_ref, r_ref, gamma_ref, beta_ref, out_ref):
    out_ref[...] = _combine_bn(agg_ref, r_ref, gamma_ref, beta_ref)


_tc_post = pl.pallas_call(
    _tc_post_body,
    out_shape=jax.ShapeDtypeStruct((N, D), jnp.float32),
)


_SC_MESH = plsc.VectorSubcoreMesh(core_axis_name="c", subcore_axis_name="s")


@functools.partial(
    pl.kernel,
    out_type=jax.ShapeDtypeStruct((NPAD, NC * MW), jnp.float32),
    mesh=_SC_MESH,
    compiler_params=pltpu.CompilerParams(use_tc_tiling_on_sc=False,
                                         needs_layout_passes=False),
    scratch_types=[
        pltpu.VMEM((2, SUP, CHUNK), jnp.int32),
        pltpu.VMEM((2, SUP, CHUNK), jnp.int32),
        pltpu.VMEM((2, KG, SUP, CHUNK), jnp.float32),
        pltpu.VMEM((3, CHUNK, GCOLS), jnp.float32),
        pltpu.VMEM((2, CHUNK, MW), jnp.float32),
        pltpu.VMEM_SHARED((NPAD, MW), jnp.float32),
        pltpu.SemaphoreType.DMA((2,)),
        pltpu.SemaphoreType.DMA((3,)),
        pltpu.SemaphoreType.DMA((2,)),
    ],
)
def _sc_conv(xg_hbm, srcc_hbm, dstc_hbm, w_hbm, agg_out,
             src_v, dst_v, w_v, rows_v, msg_v, agg_sh,
             sem_i, sem_g, sem_s):
    c = lax.axis_index("c")
    s = lax.axis_index("s")

    zf = jnp.zeros((LANES,), jnp.float32)
    of = jnp.ones((LANES,), jnp.float32)

    def zmsg(i, t):
        for j in range(MW // LANES):
            msg_v[0, i, pl.ds(j * LANES, LANES)] = zf
        return t

    lax.fori_loop(0, CHUNK, zmsg, 0)
    row0 = s * ROWS_SUB
    for j in range(ROWS_SUB // CHUNK):
        pltpu.sync_copy(msg_v.at[0],
                        agg_sh.at[pl.ds(row0 + j * CHUNK, CHUNK)])

    def ones_cols(i, t):
        msg_v[0, i, pl.ds(DH, LANES)] = of
        msg_v[1, i, pl.ds(DH, LANES)] = of
        return t

    lax.fori_loop(0, CHUNK, ones_cols, 0)
    plsc.subcore_barrier()

    chunk0 = s * CPT

    def load_super(sup, buf):
        sl = pl.ds(chunk0 + sup * SUP, SUP)
        pltpu.async_copy(srcc_hbm.at[sl, :], src_v.at[buf], sem_i.at[buf])
        pltpu.async_copy(dstc_hbm.at[sl, :], dst_v.at[buf], sem_i.at[buf])
        pltpu.async_copy(w_hbm.at[:, sl, :], w_v.at[buf], sem_i.at[buf])

    def wait_super(buf):
        pltpu.make_async_copy(srcc_hbm.at[pl.ds(0, SUP), :],
                              src_v.at[buf], sem_i.at[buf]).wait()
        pltpu.make_async_copy(dstc_hbm.at[pl.ds(0, SUP), :],
                              dst_v.at[buf], sem_i.at[buf]).wait()
        pltpu.make_async_copy(w_hbm.at[:, pl.ds(0, SUP), :],
                              w_v.at[buf], sem_i.at[buf]).wait()

    def gather(buf, q, slot):
        pltpu.async_copy(xg_hbm.at[src_v.at[buf, q]], rows_v.at[slot],
                         sem_g.at[slot])

    def wait_gather(slot):
        pltpu.make_async_copy(xg_hbm.at[src_v.at[0, 0]], rows_v.at[slot],
                              sem_g.at[slot]).wait()

    load_super(0, 0)

    def super_body(sup, t):
        cur = lax.rem(sup, 2)
        nxt = lax.rem(sup + 1, 2)
        wait_super(cur)

        coff = lax.broadcast(c * N, (LANES,))

        def rebase(i, t2):
            for g in range(EGROUPS):
                sl = pl.ds(g * LANES, LANES)
                src_v[cur, i, sl] = src_v[cur, i, sl] + coff
            return t2

        lax.fori_loop(0, SUP, rebase, 0)

        @pl.when(sup < NSUP - 1)
        def _():
            load_super(sup + 1, nxt)

        base_slot = lax.rem(sup * SUP, 3)
        for q in range(SUP):
            if q < 3:
                gather(cur, q, lax.rem(base_slot + q, 3))

        for q in range(SUP):
            slot = lax.rem(base_slot + q, 3)
            mb = q % 2
            wait_gather(slot)

            if q >= 2:
                pltpu.make_async_copy(
                    msg_v.at[mb], agg_sh.at[dst_v.at[cur, q]],
                    sem_s.at[mb]).wait()
            else:
                @pl.when(sup > 0)
                def _():
                    pltpu.make_async_copy(
                        msg_v.at[mb], agg_sh.at[dst_v.at[cur, q]],
                        sem_s.at[mb]).wait()

            def gbody(g, t2):
                e0 = g * LANES
                wvecs = [w_v[cur, k, q, pl.ds(e0, LANES)] for k in range(KG)]
                for i in range(LANES):
                    e = e0 + i
                    wk = [wvecs[k][i] for k in range(KG)]
                    for j in range(DH // LANES):
                        acc = None
                        for k in range(KG):
                            sl = pl.ds(k * DH + j * LANES, LANES)
                            term = rows_v[slot, e, sl] * wk[k]
                            acc = term if acc is None else acc + term
                        msg_v[mb, e, pl.ds(j * LANES, LANES)] = acc
                return t2

            lax.fori_loop(0, EGROUPS, gbody, 0)

            if q < SUP - 3:
                gather(cur, q + 3, slot)

            pltpu.async_copy(msg_v.at[mb], agg_sh.at[dst_v.at[cur, q]],
                             sem_s.at[mb], add=True)
        return t

    lax.fori_loop(0, NSUP, super_body, 0)

    for mb in range(2):
        pltpu.make_async_copy(msg_v.at[mb], agg_sh.at[dst_v.at[0, 0]],
                              sem_s.at[mb]).wait()
    plsc.subcore_barrier()
    pltpu.sync_copy(agg_sh.at[pl.ds(row0, ROWS_SUB)],
                    agg_out.at[pl.ds(row0, ROWS_SUB), pl.ds(c * MW, MW)])


def kernel(vals, edges, pseudo, g0, mu0, sigma0, root0, bias0, dense0,
           gamma0, beta0, g1, mu1, sigma1, root1, bias1, dense1, gamma1,
           beta1):
    srcc = edges[0].reshape(NCHUNKS, CHUNK)
    dstc = edges[1].reshape(NCHUNKS, CHUNK)
    pseudo_t = pseudo.T.reshape(PDIM, E // 128, 128)

    def permute(g):
        return jnp.concatenate(
            [g[:, k * D + cc * DH:k * D + (cc + 1) * DH]
             for cc in range(NC) for k in range(KG)], axis=1)

    gp0 = permute(g0)
    gp1 = permute(g1)
    xg0, r0, w0, w1 = _tc_pre(vals, gp0, root0, dense0, bias0, pseudo_t,
                              mu0, sigma0, mu1, sigma1)
    agg0 = _sc_conv(xg0.reshape(NC * N, GCOLS), srcc, dstc,
                    w0.reshape(KG, NCHUNKS, CHUNK))
    xg1, r1 = _tc_mid(agg0, r0, gamma0, beta0, gp1, root1, dense1, bias1)
    agg1 = _sc_conv(xg1.reshape(NC * N, GCOLS), srcc, dstc,
                    w1.reshape(KG, NCHUNKS, CHUNK))
    return _tc_post(agg1, r1, gamma1, beta1)


# speedup vs baseline: 8.8364x; 1.7378x over previous
import functools

import jax
import jax.numpy as jnp
from jax import lax
from jax.experimental import pallas as pl
from jax.experimental.pallas import tpu as pltpu
from jax.experimental.pallas import tpu_sc as plsc

N = 10000
E = 320000
D = 128
KG = 3
PDIM = 4
EPS = 1e-15

NC = 2
NS = 16
LANES = 16
DH = D // NC
GCOLS = KG * DH
MW = DH + LANES
CHUNK = 80
NCHUNKS = E // CHUNK
CPT = NCHUNKS // NS
SUP = 10
NSUP = CPT // SUP
NPAD = 10240
ROWS_SUB = NPAD // NS
EGROUPS = CHUNK // LANES


def _tc_pre_body(vals_ref, gp_ref, root_ref, dense_ref, bias_ref, pseudo_ref,
                 mu0_ref, s0_ref, mu1_ref, s1_ref,
                 xg_ref, r_ref, w0_ref, w1_ref):
    x = vals_ref[...]
    xgfull = jnp.dot(x, gp_ref[...],
                     preferred_element_type=jnp.float32).astype(jnp.bfloat16)
    xg_ref[0] = xgfull[:, 0:GCOLS]
    xg_ref[1] = xgfull[:, GCOLS:2 * GCOLS]
    r_ref[...] = (jnp.dot(x, root_ref[...] + dense_ref[...],
                          preferred_element_type=jnp.float32) + bias_ref[...])
    for mu_ref, s_ref, w_ref in ((mu0_ref, s0_ref, w0_ref),
                                 (mu1_ref, s1_ref, w1_ref)):
        mu = mu_ref[...]
        sg = s_ref[...]
        for k in range(KG):
            acc = None
            for dd in range(PDIM):
                pd = pseudo_ref[dd]
                mkd = mu[k:k + 1, dd:dd + 1]
                skd = sg[k:k + 1, dd:dd + 1]
                t = (pd - mkd) ** 2 * (-0.5 / (EPS + skd * skd))
                acc = t if acc is None else acc + t
            w_ref[k] = jnp.exp(acc)


_tc_pre = pl.pallas_call(
    _tc_pre_body,
    out_shape=[
        jax.ShapeDtypeStruct((NC, N, GCOLS), jnp.bfloat16),
        jax.ShapeDtypeStruct((N, D), jnp.float32),
        jax.ShapeDtypeStruct((KG, E // 128, 128), jnp.float32),
        jax.ShapeDtypeStruct((KG, E // 128, 128), jnp.float32),
    ],
)


def _combine_bn(agg_ref, r_ref, gamma_ref, beta_ref):
    feat = jnp.concatenate(
        [agg_ref[0:N, 0:DH], agg_ref[0:N, MW:MW + DH]], axis=1)
    cnt = agg_ref[0:N, DH:DH + 1]
    y = feat / jnp.maximum(cnt, 1.0) + r_ref[...]
    m = jnp.mean(y, axis=0, keepdims=True)
    v = jnp.mean((y - m) ** 2, axis=0, keepdims=True)
    return gamma_ref[...] * ((y - m) / jnp.sqrt(v + 1e-5)) + beta_ref[...]


def _tc_mid_body(agg_ref, r_ref, gamma_ref, beta_ref,
                 gp_ref, root_ref, dense_ref, bias_ref, xg_ref, rout_ref):
    y = _combine_bn(agg_ref, r_ref, gamma_ref, beta_ref)
    x1 = jnp.maximum(y, 0.0)
    xgfull = jnp.dot(x1, gp_ref[...],
                     preferred_element_type=jnp.float32).astype(jnp.bfloat16)
    xg_ref[0] = xgfull[:, 0:GCOLS]
    xg_ref[1] = xgfull[:, GCOLS:2 * GCOLS]
    rout_ref[...] = (jnp.dot(x1, root_ref[...] + dense_ref[...],
                             preferred_element_type=jnp.float32) + bias_ref[...])


_tc_mid = pl.pallas_call(
    _tc_mid_body,
    out_shape=[
        jax.ShapeDtypeStruct((NC, N, GCOLS), jnp.bfloat16),
        jax.ShapeDtypeStruct((N, D), jnp.float32),
    ],
)


def _tc_post_body(agg_ref, r_ref, gamma_ref, beta_ref, out_ref):
    out_ref[...] = _combine_bn(agg_ref, r_ref, gamma_ref, beta_ref)


_tc_post = pl.pallas_call(
    _tc_post_body,
    out_shape=jax.ShapeDtypeStruct((N, D), jnp.float32),
)


_SC_MESH = plsc.VectorSubcoreMesh(core_axis_name="c", subcore_axis_name="s")


@functools.partial(
    pl.kernel,
    out_type=jax.ShapeDtypeStruct((NPAD, NC * MW), jnp.float32),
    mesh=_SC_MESH,
    compiler_params=pltpu.CompilerParams(use_tc_tiling_on_sc=False,
                                         needs_layout_passes=False),
    scratch_types=[
        pltpu.VMEM((2, SUP, CHUNK), jnp.int32),
        pltpu.VMEM((2, SUP, CHUNK), jnp.int32),
        pltpu.VMEM((2, KG, SUP, CHUNK), jnp.float32),
        pltpu.VMEM((5, CHUNK, GCOLS), jnp.bfloat16),
        pltpu.VMEM((2, CHUNK, MW), jnp.float32),
        pltpu.VMEM_SHARED((NPAD, MW), jnp.float32),
        pltpu.SemaphoreType.DMA((2,)),
        pltpu.SemaphoreType.DMA((5,)),
        pltpu.SemaphoreType.DMA((2,)),
    ],
)
def _sc_conv(xg_hbm, srcc_hbm, dstc_hbm, w_hbm, agg_out,
             src_v, dst_v, w_v, rows_v, msg_v, agg_sh,
             sem_i, sem_g, sem_s):
    c = lax.axis_index("c")
    s = lax.axis_index("s")

    zf = jnp.zeros((LANES,), jnp.float32)
    of = jnp.ones((LANES,), jnp.float32)

    def zmsg(i, t):
        for j in range(MW // LANES):
            msg_v[0, i, pl.ds(j * LANES, LANES)] = zf
        return t

    lax.fori_loop(0, CHUNK, zmsg, 0)
    row0 = s * ROWS_SUB
    for j in range(ROWS_SUB // CHUNK):
        pltpu.sync_copy(msg_v.at[0],
                        agg_sh.at[pl.ds(row0 + j * CHUNK, CHUNK)])

    def ones_cols(i, t):
        msg_v[0, i, pl.ds(DH, LANES)] = of
        msg_v[1, i, pl.ds(DH, LANES)] = of
        return t

    lax.fori_loop(0, CHUNK, ones_cols, 0)
    plsc.subcore_barrier()

    chunk0 = s * CPT

    def load_super(sup, buf):
        sl = pl.ds(chunk0 + sup * SUP, SUP)
        pltpu.async_copy(srcc_hbm.at[sl, :], src_v.at[buf], sem_i.at[buf])
        pltpu.async_copy(dstc_hbm.at[sl, :], dst_v.at[buf], sem_i.at[buf])
        pltpu.async_copy(w_hbm.at[:, sl, :], w_v.at[buf], sem_i.at[buf])

    def wait_super(buf):
        pltpu.make_async_copy(srcc_hbm.at[pl.ds(0, SUP), :],
                              src_v.at[buf], sem_i.at[buf]).wait()
        pltpu.make_async_copy(dstc_hbm.at[pl.ds(0, SUP), :],
                              dst_v.at[buf], sem_i.at[buf]).wait()
        pltpu.make_async_copy(w_hbm.at[:, pl.ds(0, SUP), :],
                              w_v.at[buf], sem_i.at[buf]).wait()

    def gather(buf, q, slot):
        pltpu.async_copy(xg_hbm.at[src_v.at[buf, q]], rows_v.at[slot],
                         sem_g.at[slot])

    def wait_gather(slot):
        pltpu.make_async_copy(xg_hbm.at[src_v.at[0, 0]], rows_v.at[slot],
                              sem_g.at[slot]).wait()

    load_super(0, 0)

    def super_body(sup, t):
        cur = lax.rem(sup, 2)
        nxt = lax.rem(sup + 1, 2)
        wait_super(cur)

        coff = lax.broadcast(c * N, (LANES,))

        def rebase(i, t2):
            for g in range(EGROUPS):
                sl = pl.ds(g * LANES, LANES)
                src_v[cur, i, sl] = src_v[cur, i, sl] + coff
            return t2

        lax.fori_loop(0, SUP, rebase, 0)

        @pl.when(sup < NSUP - 1)
        def _():
            load_super(sup + 1, nxt)

        for q in range(SUP):
            if q < 5:
                gather(cur, q, q % 5)

        for q in range(SUP):
            slot = q % 5
            mb = q % 2
            wait_gather(slot)

            if q >= 2:
                pltpu.make_async_copy(
                    msg_v.at[mb], agg_sh.at[dst_v.at[cur, q]],
                    sem_s.at[mb]).wait()
            else:
                @pl.when(sup > 0)
                def _():
                    pltpu.make_async_copy(
                        msg_v.at[mb], agg_sh.at[dst_v.at[cur, q]],
                        sem_s.at[mb]).wait()

            def gbody(g, t2):
                e0 = g * LANES
                wvecs = [w_v[cur, k, q, pl.ds(e0, LANES)] for k in range(KG)]
                for i in range(LANES):
                    e = e0 + i
                    wk = [wvecs[k][i] for k in range(KG)]
                    accs = [None] * (DH // LANES)
                    for k in range(KG):
                        for half in range(2):
                            x32 = rows_v[slot, e,
                                         pl.ds(k * DH + half * 32, 32)]
                            a, b = plsc.unpack(
                                x32, format=plsc.PackFormat.INTERLEAVED)
                            for j, v in ((2 * half, a), (2 * half + 1, b)):
                                term = v * wk[k]
                                accs[j] = (term if accs[j] is None
                                           else accs[j] + term)
                    for j in range(DH // LANES):
                        msg_v[mb, e, pl.ds(j * LANES, LANES)] = accs[j]
                return t2

            lax.fori_loop(0, EGROUPS, gbody, 0)

            if q < SUP - 5:
                gather(cur, q + 5, slot)

            pltpu.async_copy(msg_v.at[mb], agg_sh.at[dst_v.at[cur, q]],
                             sem_s.at[mb], add=True)
        return t

    lax.fori_loop(0, NSUP, super_body, 0)

    for mb in range(2):
        pltpu.make_async_copy(msg_v.at[mb], agg_sh.at[dst_v.at[0, 0]],
                              sem_s.at[mb]).wait()
    plsc.subcore_barrier()
    pltpu.sync_copy(agg_sh.at[pl.ds(row0, ROWS_SUB)],
                    agg_out.at[pl.ds(row0, ROWS_SUB), pl.ds(c * MW, MW)])


def kernel(vals, edges, pseudo, g0, mu0, sigma0, root0, bias0, dense0,
           gamma0, beta0, g1, mu1, sigma1, root1, bias1, dense1, gamma1,
           beta1):
    srcc = edges[0].reshape(NCHUNKS, CHUNK)
    dstc = edges[1].reshape(NCHUNKS, CHUNK)
    pseudo_t = pseudo.T.reshape(PDIM, E // 128, 128)

    perm = []
    for cc in range(NC):
        for k in range(KG):
            base = k * D + cc * DH
            for half in range(2):
                for i in range(LANES):
                    perm.append(base + half * 32 + i)
                    perm.append(base + half * 32 + LANES + i)
    perm = jnp.array(perm, dtype=jnp.int32)

    def permute(g):
        return g[:, perm]

    gp0 = permute(g0)
    gp1 = permute(g1)
    xg0, r0, w0, w1 = _tc_pre(vals, gp0, root0, dense0, bias0, pseudo_t,
                              mu0, sigma0, mu1, sigma1)
    agg0 = _sc_conv(xg0.reshape(NC * N, GCOLS), srcc, dstc,
                    w0.reshape(KG, NCHUNKS, CHUNK))
    xg1, r1 = _tc_mid(agg0, r0, gamma0, beta0, gp1, root1, dense1, bias1)
    agg1 = _sc_conv(xg1.reshape(NC * N, GCOLS), srcc, dstc,
                    w1.reshape(KG, NCHUNKS, CHUNK))
    return _tc_post(agg1, r1, gamma1, beta1)


# speedup vs baseline: 9.5803x; 1.0842x over previous
import functools

import jax
import jax.numpy as jnp
from jax import lax
from jax.experimental import pallas as pl
from jax.experimental.pallas import tpu as pltpu
from jax.experimental.pallas import tpu_sc as plsc

N = 10000
E = 320000
D = 128
KG = 3
PDIM = 4
EPS = 1e-15

NC = 2
NS = 16
LANES = 16
DH = D // NC
GCOLS = KG * DH
MW = DH + LANES
CHUNK = 128
NCHUNKS = E // CHUNK
BASE_CPT = NCHUNKS // NS
TAIL = NCHUNKS - BASE_CPT * NS
SUP = 12
NSUP = BASE_CPT // SUP
NPAD = 10240
ROWS_SUB = NPAD // NS
EGROUPS = CHUNK // LANES


def _tc_pre_body(vals_ref, gp_ref, root_ref, dense_ref, bias_ref, pseudo_ref,
                 mu0_ref, s0_ref, mu1_ref, s1_ref,
                 xg_ref, r_ref, w0_ref, w1_ref):
    x = vals_ref[...]
    xgfull = jnp.dot(x, gp_ref[...],
                     preferred_element_type=jnp.float32).astype(jnp.bfloat16)
    xg_ref[0] = xgfull[:, 0:GCOLS]
    xg_ref[1] = xgfull[:, GCOLS:2 * GCOLS]
    r_ref[...] = (jnp.dot(x, root_ref[...] + dense_ref[...],
                          preferred_element_type=jnp.float32) + bias_ref[...])
    for mu_ref, s_ref, w_ref in ((mu0_ref, s0_ref, w0_ref),
                                 (mu1_ref, s1_ref, w1_ref)):
        mu = mu_ref[...]
        sg = s_ref[...]
        for k in range(KG):
            acc = None
            for dd in range(PDIM):
                pd = pseudo_ref[dd]
                mkd = mu[k:k + 1, dd:dd + 1]
                skd = sg[k:k + 1, dd:dd + 1]
                t = (pd - mkd) ** 2 * (-0.5 / (EPS + skd * skd))
                acc = t if acc is None else acc + t
            w_ref[k] = jnp.exp(acc)


_tc_pre = pl.pallas_call(
    _tc_pre_body,
    out_shape=[
        jax.ShapeDtypeStruct((NC, N, GCOLS), jnp.bfloat16),
        jax.ShapeDtypeStruct((N, D), jnp.float32),
        jax.ShapeDtypeStruct((KG, E // 128, 128), jnp.float32),
        jax.ShapeDtypeStruct((KG, E // 128, 128), jnp.float32),
    ],
)


def _combine_bn(agg_ref, r_ref, gamma_ref, beta_ref):
    feat = jnp.concatenate(
        [agg_ref[0:N, 0:DH], agg_ref[0:N, MW:MW + DH]], axis=1)
    cnt = agg_ref[0:N, DH:DH + 1]
    y = feat / jnp.maximum(cnt, 1.0) + r_ref[...]
    m = jnp.mean(y, axis=0, keepdims=True)
    v = jnp.mean((y - m) ** 2, axis=0, keepdims=True)
    return gamma_ref[...] * ((y - m) / jnp.sqrt(v + 1e-5)) + beta_ref[...]


def _tc_mid_body(agg_ref, r_ref, gamma_ref, beta_ref,
                 gp_ref, root_ref, dense_ref, bias_ref, xg_ref, rout_ref):
    y = _combine_bn(agg_ref, r_ref, gamma_ref, beta_ref)
    x1 = jnp.maximum(y, 0.0)
    xgfull = jnp.dot(x1, gp_ref[...],
                     preferred_element_type=jnp.float32).astype(jnp.bfloat16)
    xg_ref[0] = xgfull[:, 0:GCOLS]
    xg_ref[1] = xgfull[:, GCOLS:2 * GCOLS]
    rout_ref[...] = (jnp.dot(x1, root_ref[...] + dense_ref[...],
                             preferred_element_type=jnp.float32) + bias_ref[...])


_tc_mid = pl.pallas_call(
    _tc_mid_body,
    out_shape=[
        jax.ShapeDtypeStruct((NC, N, GCOLS), jnp.bfloat16),
        jax.ShapeDtypeStruct((N, D), jnp.float32),
    ],
)


def _tc_post_body(agg_ref, r_ref, gamma_ref, beta_ref, out_ref):
    out_ref[...] = _combine_bn(agg_ref, r_ref, gamma_ref, beta_ref)


_tc_post = pl.pallas_call(
    _tc_post_body,
    out_shape=jax.ShapeDtypeStruct((N, D), jnp.float32),
)


_SC_MESH = plsc.VectorSubcoreMesh(core_axis_name="c", subcore_axis_name="s")


@functools.partial(
    pl.kernel,
    out_type=jax.ShapeDtypeStruct((NPAD, NC * MW), jnp.float32),
    mesh=_SC_MESH,
    compiler_params=pltpu.CompilerParams(use_tc_tiling_on_sc=False,
                                         needs_layout_passes=False),
    scratch_types=[
        pltpu.VMEM((2, SUP, CHUNK), jnp.int32),
        pltpu.VMEM((2, SUP, CHUNK), jnp.int32),
        pltpu.VMEM((2, KG, SUP, CHUNK), jnp.float32),
        pltpu.VMEM((3, CHUNK, GCOLS), jnp.bfloat16),
        pltpu.VMEM((2, CHUNK, MW), jnp.float32),
        pltpu.VMEM_SHARED((NPAD, MW), jnp.float32),
        pltpu.SemaphoreType.DMA((2,)),
        pltpu.SemaphoreType.DMA((3,)),
        pltpu.SemaphoreType.DMA((2,)),
    ],
)
def _sc_conv(xg_hbm, srcc_hbm, dstc_hbm, w_hbm, agg_out,
             src_v, dst_v, w_v, rows_v, msg_v, agg_sh,
             sem_i, sem_g, sem_s):
    c = lax.axis_index("c")
    s = lax.axis_index("s")

    zf = jnp.zeros((LANES,), jnp.float32)
    of = jnp.ones((LANES,), jnp.float32)

    def zmsg(i, t):
        for j in range(MW // LANES):
            msg_v[0, i, pl.ds(j * LANES, LANES)] = zf
        return t

    lax.fori_loop(0, CHUNK, zmsg, 0)
    row0 = s * ROWS_SUB
    for j in range(ROWS_SUB // CHUNK):
        pltpu.sync_copy(msg_v.at[0],
                        agg_sh.at[pl.ds(row0 + j * CHUNK, CHUNK)])

    def ones_cols(i, t):
        msg_v[0, i, pl.ds(DH, LANES)] = of
        msg_v[1, i, pl.ds(DH, LANES)] = of
        return t

    lax.fori_loop(0, CHUNK, ones_cols, 0)
    plsc.subcore_barrier()

    chunk0 = s * BASE_CPT

    def load_super(sup, buf):
        sl = pl.ds(chunk0 + sup * SUP, SUP)
        pltpu.async_copy(srcc_hbm.at[sl, :], src_v.at[buf], sem_i.at[buf])
        pltpu.async_copy(dstc_hbm.at[sl, :], dst_v.at[buf], sem_i.at[buf])
        pltpu.async_copy(w_hbm.at[:, sl, :], w_v.at[buf], sem_i.at[buf])

    def wait_super(buf):
        pltpu.make_async_copy(srcc_hbm.at[pl.ds(0, SUP), :],
                              src_v.at[buf], sem_i.at[buf]).wait()
        pltpu.make_async_copy(dstc_hbm.at[pl.ds(0, SUP), :],
                              dst_v.at[buf], sem_i.at[buf]).wait()
        pltpu.make_async_copy(w_hbm.at[:, pl.ds(0, SUP), :],
                              w_v.at[buf], sem_i.at[buf]).wait()

    def gather(buf, q, slot):
        pltpu.async_copy(xg_hbm.at[src_v.at[buf, q]], rows_v.at[slot],
                         sem_g.at[slot])

    def wait_gather(slot):
        pltpu.make_async_copy(xg_hbm.at[src_v.at[0, 0]], rows_v.at[slot],
                              sem_g.at[slot]).wait()

    load_super(0, 0)

    def super_body(sup, t):
        cur = lax.rem(sup, 2)
        nxt = lax.rem(sup + 1, 2)
        wait_super(cur)

        coff = lax.broadcast(c * N, (LANES,))

        def rebase(i, t2):
            for g in range(EGROUPS):
                sl = pl.ds(g * LANES, LANES)
                src_v[cur, i, sl] = src_v[cur, i, sl] + coff
            return t2

        lax.fori_loop(0, SUP, rebase, 0)

        @pl.when(sup < NSUP - 1)
        def _():
            load_super(sup + 1, nxt)

        for q in range(SUP):
            if q < 3:
                gather(cur, q, q % 3)

        for q in range(SUP):
            slot = q % 3
            mb = q % 2
            wait_gather(slot)

            if q >= 2:
                pltpu.make_async_copy(
                    msg_v.at[mb], agg_sh.at[dst_v.at[cur, q]],
                    sem_s.at[mb]).wait()
            else:
                @pl.when(sup > 0)
                def _():
                    pltpu.make_async_copy(
                        msg_v.at[mb], agg_sh.at[dst_v.at[cur, q]],
                        sem_s.at[mb]).wait()

            compute_msg(cur, q, slot, mb)

            if q < SUP - 3:
                gather(cur, q + 3, slot)

            pltpu.async_copy(msg_v.at[mb], agg_sh.at[dst_v.at[cur, q]],
                             sem_s.at[mb], add=True)
        return t

    def compute_msg(buf, q, slot, mb):
        def gbody(g, t2):
            e0 = g * LANES
            wvecs = [w_v[buf, k, q, pl.ds(e0, LANES)] for k in range(KG)]
            for i in range(LANES):
                e = e0 + i
                wk = [wvecs[k][i] for k in range(KG)]
                accs = [None] * (DH // LANES)
                for k in range(KG):
                    for half in range(2):
                        x32 = rows_v[slot, e,
                                     pl.ds(k * DH + half * 32, 32)]
                        a, b = plsc.unpack(
                            x32, format=plsc.PackFormat.INTERLEAVED)
                        for j, v in ((2 * half, a), (2 * half + 1, b)):
                            term = v * wk[k]
                            accs[j] = (term if accs[j] is None
                                       else accs[j] + term)
                for j in range(DH // LANES):
                    msg_v[mb, e, pl.ds(j * LANES, LANES)] = accs[j]
            return t2

        lax.fori_loop(0, EGROUPS, gbody, 0)

    lax.fori_loop(0, NSUP, super_body, 0)

    for mb in range(2):
        pltpu.make_async_copy(msg_v.at[mb], agg_sh.at[dst_v.at[0, 0]],
                              sem_s.at[mb]).wait()

    @pl.when(s < TAIL)
    def _():
        tc = NS * BASE_CPT + s
        pltpu.sync_copy(srcc_hbm.at[pl.ds(tc, 1), :],
                        src_v.at[0, pl.ds(0, 1), :])
        pltpu.sync_copy(dstc_hbm.at[pl.ds(tc, 1), :],
                        dst_v.at[0, pl.ds(0, 1), :])
        pltpu.sync_copy(w_hbm.at[:, pl.ds(tc, 1), :],
                        w_v.at[0, :, pl.ds(0, 1), :])
        coff = lax.broadcast(c * N, (LANES,))
        for g in range(EGROUPS):
            sl = pl.ds(g * LANES, LANES)
            src_v[0, 0, sl] = src_v[0, 0, sl] + coff
        pltpu.async_copy(xg_hbm.at[src_v.at[0, 0]], rows_v.at[0],
                         sem_g.at[0])
        pltpu.make_async_copy(xg_hbm.at[src_v.at[0, 0]], rows_v.at[0],
                              sem_g.at[0]).wait()
        compute_msg(0, 0, 0, 0)
        pltpu.sync_copy(msg_v.at[0], agg_sh.at[dst_v.at[0, 0]], add=True)

    plsc.subcore_barrier()
    pltpu.sync_copy(agg_sh.at[pl.ds(row0, ROWS_SUB)],
                    agg_out.at[pl.ds(row0, ROWS_SUB), pl.ds(c * MW, MW)])


def kernel(vals, edges, pseudo, g0, mu0, sigma0, root0, bias0, dense0,
           gamma0, beta0, g1, mu1, sigma1, root1, bias1, dense1, gamma1,
           beta1):
    srcc = edges[0].reshape(NCHUNKS, CHUNK)
    dstc = edges[1].reshape(NCHUNKS, CHUNK)
    pseudo_t = pseudo.T.reshape(PDIM, E // 128, 128)

    perm = []
    for cc in range(NC):
        for k in range(KG):
            base = k * D + cc * DH
            for half in range(2):
                for i in range(LANES):
                    perm.append(base + half * 32 + i)
                    perm.append(base + half * 32 + LANES + i)
    perm = jnp.array(perm, dtype=jnp.int32)

    def permute(g):
        return g[:, perm]

    gp0 = permute(g0)
    gp1 = permute(g1)
    xg0, r0, w0, w1 = _tc_pre(vals, gp0, root0, dense0, bias0, pseudo_t,
                              mu0, sigma0, mu1, sigma1)
    agg0 = _sc_conv(xg0.reshape(NC * N, GCOLS), srcc, dstc,
                    w0.reshape(KG, NCHUNKS, CHUNK))
    xg1, r1 = _tc_mid(agg0, r0, gamma0, beta0, gp1, root1, dense1, bias1)
    agg1 = _sc_conv(xg1.reshape(NC * N, GCOLS), srcc, dstc,
                    w1.reshape(KG, NCHUNKS, CHUNK))
    return _tc_post(agg1, r1, gamma1, beta1)
